# Initial kernel scaffold; baseline (speedup 1.0000x reference)
import jax
import jax.numpy as jnp
from jax.experimental import pallas as pl


def kernel(x, t_day, t_week, W_tok, b_tok, day_table, week_table, W_sp, b_sp):
    raise NotImplementedError("write your pallas kernel here")


# speedup vs baseline: 1.0586x; 1.0586x over previous
import functools

import jax
import jax.numpy as jnp
from jax import lax
from jax.experimental import pallas as pl
from jax.experimental.pallas import tpu as pltpu
from jax.experimental.pallas import tpu_sc as plsc

_B, _T, _N, _DIN, _H = 16, 12, 2048, 3, 32
_BT = _B * _T
_LANES = 16


def _sc_bias_rows(idx_d, idx_w, day_pad, week_pad, b_cat):
    info = plsc.get_sparse_core_info()
    nw = info.num_cores * info.num_subcores
    chunk = 8 * nw
    pad_bt = ((_BT + chunk - 1) // chunk) * chunk
    rows_per_w = pad_bt // nw
    idx_d = jnp.pad(idx_d, (0, pad_bt - _BT))
    idx_w = jnp.pad(idx_w, (0, pad_bt - _BT))
    _C = 4 * _H

    def body(idx_d_hbm, idx_w_hbm, day_hbm, week_hbm, bcat_hbm,
             out_hbm, idxd_v, idxw_v, dayrows_v, weekrows_v,
             bcat_v, comb_v, sem_d, sem_w):
        nc = info.num_cores
        wid = lax.axis_index("s") * nc + lax.axis_index("c")
        base = wid * rows_per_w
        pltpu.sync_copy(idx_d_hbm.at[pl.ds(base, rows_per_w)], idxd_v)
        pltpu.sync_copy(idx_w_hbm.at[pl.ds(base, rows_per_w)], idxw_v)
        pltpu.sync_copy(bcat_hbm, bcat_v)
        pltpu.async_copy(day_hbm.at[idxd_v], dayrows_v, sem_d).wait()
        pltpu.async_copy(week_hbm.at[idxw_v], weekrows_v, sem_w).wait()
        for r in range(rows_per_w):
            for c in range(_C // _LANES):
                sl = pl.ds(c * _LANES, _LANES)
                comb_v[r, sl] = bcat_v[sl] + dayrows_v[r, sl] + weekrows_v[r, sl]
        pltpu.sync_copy(comb_v, out_hbm.at[pl.ds(base, rows_per_w)])

    fn = functools.partial(
        pl.kernel,
        mesh=plsc.VectorSubcoreMesh(core_axis_name="c", subcore_axis_name="s"),
        out_type=jax.ShapeDtypeStruct((pad_bt, _C), jnp.float32),
        scratch_types=[
            pltpu.VMEM((rows_per_w,), jnp.int32),
            pltpu.VMEM((rows_per_w,), jnp.int32),
            pltpu.VMEM((rows_per_w, _C), jnp.float32),
            pltpu.VMEM((rows_per_w, _C), jnp.float32),
            pltpu.VMEM((_C,), jnp.float32),
            pltpu.VMEM((rows_per_w, _C), jnp.float32),
            pltpu.SemaphoreType.DMA,
            pltpu.SemaphoreType.DMA,
        ],
    )(body)
    return fn(idx_d, idx_w, day_pad, week_pad, b_cat)


def _tc_body(x_ref, w_ref, b_ref, o_ref):
    xb = x_ref[0]
    h = jnp.dot(xb, w_ref[...], preferred_element_type=jnp.float32)
    v = h + b_ref[0]
    lane = lax.broadcasted_iota(jnp.int32, v.shape, 1)
    o_ref[0] = jnp.where(lane >= 3 * _H, jnp.maximum(v, 0.0), v)


def _tc_fused(x2, w_cat, bias3):
    return pl.pallas_call(
        _tc_body,
        grid=(_BT,),
        in_specs=[
            pl.BlockSpec((1, _N, _DIN), lambda i: (i, 0, 0)),
            pl.BlockSpec((_DIN, 4 * _H), lambda i: (0, 0)),
            pl.BlockSpec((1, 1, 4 * _H), lambda i: (i, 0, 0)),
        ],
        out_specs=pl.BlockSpec((1, _N, 4 * _H), lambda i: (i, 0, 0)),
        out_shape=jax.ShapeDtypeStruct((_BT, _N, 4 * _H), jnp.float32),
    )(x2, w_cat, bias3)


def kernel(x, t_day, t_week, W_tok, b_tok, day_table, week_table, W_sp, b_sp):
    idx_d = t_day.reshape(-1).astype(jnp.int32)
    idx_w = t_week.reshape(-1).astype(jnp.int32)
    day_pad = (jnp.zeros((day_table.shape[0], 4 * _H), jnp.float32)
               .at[:, _H : 2 * _H].set(day_table))
    week_pad = (jnp.zeros((week_table.shape[0], 4 * _H), jnp.float32)
                .at[:, 2 * _H : 3 * _H].set(week_table))
    b_cat = jnp.concatenate([b_tok, jnp.zeros((2 * _H,), jnp.float32), b_sp])
    bias = _sc_bias_rows(idx_d, idx_w, day_pad, week_pad, b_cat)
    bias3 = bias[:_BT].reshape(_BT, 1, 4 * _H)
    w_cat = (jnp.zeros((_DIN, 4 * _H), jnp.float32)
             .at[:, : _H].set(W_tok.T)
             .at[:, 3 * _H :].set(W_sp.T))
    x2 = x.reshape(_BT, _N, _DIN)
    out = _tc_fused(x2, w_cat, bias3)
    return out.reshape(_B, _T, _N, 4 * _H)


# speedup vs baseline: 1.3779x; 1.3016x over previous
import functools

import jax
import jax.numpy as jnp
from jax import lax
from jax.experimental import pallas as pl
from jax.experimental.pallas import tpu as pltpu
from jax.experimental.pallas import tpu_sc as plsc

_B, _T, _N, _DIN, _H = 16, 12, 2048, 3, 32
_BT = _B * _T
_LANES = 16


def _sc_bias_rows(idx_d, idx_w, day_pad, week_pad, b_cat):
    info = plsc.get_sparse_core_info()
    nw = info.num_cores * info.num_subcores
    chunk = 8 * nw
    pad_bt = ((_BT + chunk - 1) // chunk) * chunk
    rows_per_w = pad_bt // nw
    idx_d = jnp.pad(idx_d, (0, pad_bt - _BT))
    idx_w = jnp.pad(idx_w, (0, pad_bt - _BT))
    _C = 4 * _H

    def body(idx_d_hbm, idx_w_hbm, day_hbm, week_hbm, bcat_hbm,
             out_hbm, idxd_v, idxw_v, dayrows_v, weekrows_v,
             bcat_v, comb_v, sem_d, sem_w):
        nc = info.num_cores
        wid = lax.axis_index("s") * nc + lax.axis_index("c")
        base = wid * rows_per_w
        pltpu.sync_copy(idx_d_hbm.at[pl.ds(base, rows_per_w)], idxd_v)
        pltpu.sync_copy(idx_w_hbm.at[pl.ds(base, rows_per_w)], idxw_v)
        pltpu.sync_copy(bcat_hbm, bcat_v)
        pltpu.async_copy(day_hbm.at[idxd_v], dayrows_v, sem_d).wait()
        pltpu.async_copy(week_hbm.at[idxw_v], weekrows_v, sem_w).wait()
        for r in range(rows_per_w):
            for c in range(_C // _LANES):
                sl = pl.ds(c * _LANES, _LANES)
                comb_v[r, sl] = bcat_v[sl] + dayrows_v[r, sl] + weekrows_v[r, sl]
        pltpu.sync_copy(comb_v, out_hbm.at[pl.ds(base, rows_per_w)])

    fn = functools.partial(
        pl.kernel,
        mesh=plsc.VectorSubcoreMesh(core_axis_name="c", subcore_axis_name="s"),
        out_type=jax.ShapeDtypeStruct((pad_bt, _C), jnp.float32),
        scratch_types=[
            pltpu.VMEM((rows_per_w,), jnp.int32),
            pltpu.VMEM((rows_per_w,), jnp.int32),
            pltpu.VMEM((rows_per_w, _C), jnp.float32),
            pltpu.VMEM((rows_per_w, _C), jnp.float32),
            pltpu.VMEM((_C,), jnp.float32),
            pltpu.VMEM((rows_per_w, _C), jnp.float32),
            pltpu.SemaphoreType.DMA,
            pltpu.SemaphoreType.DMA,
        ],
    )(body)
    return fn(idx_d, idx_w, day_pad, week_pad, b_cat)


_G = 4


def _tc_body(x_ref, w_ref, b_ref, o_ref):
    for g in range(_G):
        xb = x_ref[g]
        h = jnp.dot(xb, w_ref[...], preferred_element_type=jnp.float32)
        v = h + b_ref[g]
        lane = lax.broadcasted_iota(jnp.int32, v.shape, 1)
        o_ref[g] = jnp.where(lane >= 3 * _H, jnp.maximum(v, 0.0), v)


def _tc_fused(x2, w_cat, bias3):
    return pl.pallas_call(
        _tc_body,
        grid=(_BT // _G,),
        in_specs=[
            pl.BlockSpec((_G, _N, _DIN), lambda i: (i, 0, 0)),
            pl.BlockSpec((_DIN, 4 * _H), lambda i: (0, 0)),
            pl.BlockSpec((_G, 1, 4 * _H), lambda i: (i, 0, 0)),
        ],
        out_specs=pl.BlockSpec((_G, _N, 4 * _H), lambda i: (i, 0, 0)),
        out_shape=jax.ShapeDtypeStruct((_BT, _N, 4 * _H), jnp.float32),
    )(x2, w_cat, bias3)


def kernel(x, t_day, t_week, W_tok, b_tok, day_table, week_table, W_sp, b_sp):
    idx_d = t_day.reshape(-1).astype(jnp.int32)
    idx_w = t_week.reshape(-1).astype(jnp.int32)
    day_pad = (jnp.zeros((day_table.shape[0], 4 * _H), jnp.float32)
               .at[:, _H : 2 * _H].set(day_table))
    week_pad = (jnp.zeros((week_table.shape[0], 4 * _H), jnp.float32)
                .at[:, 2 * _H : 3 * _H].set(week_table))
    b_cat = jnp.concatenate([b_tok, jnp.zeros((2 * _H,), jnp.float32), b_sp])
    bias = _sc_bias_rows(idx_d, idx_w, day_pad, week_pad, b_cat)
    bias3 = bias[:_BT].reshape(_BT, 1, 4 * _H)
    w_cat = (jnp.zeros((_DIN, 4 * _H), jnp.float32)
             .at[:, : _H].set(W_tok.T)
             .at[:, 3 * _H :].set(W_sp.T))
    x2 = x.reshape(_BT, _N, _DIN)
    out = _tc_fused(x2, w_cat, bias3)
    return out.reshape(_B, _T, _N, 4 * _H)


# speedup vs baseline: 1.3783x; 1.0003x over previous
import functools

import jax
import jax.numpy as jnp
from jax import lax
from jax.experimental import pallas as pl
from jax.experimental.pallas import tpu as pltpu
from jax.experimental.pallas import tpu_sc as plsc

_B, _T, _N, _DIN, _H = 16, 12, 2048, 3, 32
_BT = _B * _T
_LANES = 16


def _sc_bias_rows(idx_d, idx_w, day_pad, week_pad, b_cat):
    info = plsc.get_sparse_core_info()
    nw = info.num_cores * info.num_subcores
    chunk = 8 * nw
    pad_bt = ((_BT + chunk - 1) // chunk) * chunk
    rows_per_w = pad_bt // nw
    idx_d = jnp.pad(idx_d, (0, pad_bt - _BT))
    idx_w = jnp.pad(idx_w, (0, pad_bt - _BT))
    _C = 4 * _H

    def body(idx_d_hbm, idx_w_hbm, day_hbm, week_hbm, bcat_hbm,
             out_hbm, idxd_v, idxw_v, dayrows_v, weekrows_v,
             bcat_v, comb_v, sem_d, sem_w):
        nc = info.num_cores
        wid = lax.axis_index("s") * nc + lax.axis_index("c")
        base = wid * rows_per_w
        pltpu.sync_copy(idx_d_hbm.at[pl.ds(base, rows_per_w)], idxd_v)
        pltpu.sync_copy(idx_w_hbm.at[pl.ds(base, rows_per_w)], idxw_v)
        pltpu.sync_copy(bcat_hbm, bcat_v)
        pltpu.async_copy(day_hbm.at[idxd_v], dayrows_v, sem_d).wait()
        pltpu.async_copy(week_hbm.at[idxw_v], weekrows_v, sem_w).wait()
        for r in range(rows_per_w):
            for c in range(_C // _LANES):
                sl = pl.ds(c * _LANES, _LANES)
                comb_v[r, sl] = bcat_v[sl] + dayrows_v[r, sl] + weekrows_v[r, sl]
        pltpu.sync_copy(comb_v, out_hbm.at[pl.ds(base, rows_per_w)])

    fn = functools.partial(
        pl.kernel,
        mesh=plsc.VectorSubcoreMesh(core_axis_name="c", subcore_axis_name="s"),
        out_type=jax.ShapeDtypeStruct((pad_bt, _C), jnp.float32),
        scratch_types=[
            pltpu.VMEM((rows_per_w,), jnp.int32),
            pltpu.VMEM((rows_per_w,), jnp.int32),
            pltpu.VMEM((rows_per_w, _C), jnp.float32),
            pltpu.VMEM((rows_per_w, _C), jnp.float32),
            pltpu.VMEM((_C,), jnp.float32),
            pltpu.VMEM((rows_per_w, _C), jnp.float32),
            pltpu.SemaphoreType.DMA,
            pltpu.SemaphoreType.DMA,
        ],
    )(body)
    return fn(idx_d, idx_w, day_pad, week_pad, b_cat)


_G = 8


def _tc_body(x_ref, w_ref, b_ref, o_ref):
    for g in range(_G):
        xb = x_ref[g]
        h = jnp.dot(xb, w_ref[...], preferred_element_type=jnp.float32)
        v = h + b_ref[g]
        lane = lax.broadcasted_iota(jnp.int32, v.shape, 1)
        o_ref[g] = jnp.where(lane >= 3 * _H, jnp.maximum(v, 0.0), v)


def _tc_fused(x2, w_cat, bias3):
    return pl.pallas_call(
        _tc_body,
        grid=(_BT // _G,),
        in_specs=[
            pl.BlockSpec((_G, _N, _DIN), lambda i: (i, 0, 0)),
            pl.BlockSpec((_DIN, 4 * _H), lambda i: (0, 0)),
            pl.BlockSpec((_G, 1, 4 * _H), lambda i: (i, 0, 0)),
        ],
        out_specs=pl.BlockSpec((_G, _N, 4 * _H), lambda i: (i, 0, 0)),
        out_shape=jax.ShapeDtypeStruct((_BT, _N, 4 * _H), jnp.float32),
    )(x2, w_cat, bias3)


def kernel(x, t_day, t_week, W_tok, b_tok, day_table, week_table, W_sp, b_sp):
    idx_d = t_day.reshape(-1).astype(jnp.int32)
    idx_w = t_week.reshape(-1).astype(jnp.int32)
    day_pad = (jnp.zeros((day_table.shape[0], 4 * _H), jnp.float32)
               .at[:, _H : 2 * _H].set(day_table))
    week_pad = (jnp.zeros((week_table.shape[0], 4 * _H), jnp.float32)
                .at[:, 2 * _H : 3 * _H].set(week_table))
    b_cat = jnp.concatenate([b_tok, jnp.zeros((2 * _H,), jnp.float32), b_sp])
    bias = _sc_bias_rows(idx_d, idx_w, day_pad, week_pad, b_cat)
    bias3 = bias[:_BT].reshape(_BT, 1, 4 * _H)
    w_cat = (jnp.zeros((_DIN, 4 * _H), jnp.float32)
             .at[:, : _H].set(W_tok.T)
             .at[:, 3 * _H :].set(W_sp.T))
    x2 = x.reshape(_BT, _N, _DIN)
    out = _tc_fused(x2, w_cat, bias3)
    return out.reshape(_B, _T, _N, 4 * _H)


# speedup vs baseline: 3.2553x; 2.3618x over previous
import functools

import jax
import jax.numpy as jnp
from jax import lax
from jax.experimental import pallas as pl
from jax.experimental.pallas import tpu as pltpu
from jax.experimental.pallas import tpu_sc as plsc

_B, _T, _N, _DIN, _H = 16, 12, 2048, 3, 32
_BT = _B * _T
_LANES = 16


def _sc_bias_rows(idx_d, idx_w, day_pad, week_pad, b_cat):
    info = plsc.get_sparse_core_info()
    nw = info.num_cores * info.num_subcores
    chunk = 8 * nw
    pad_bt = ((_BT + chunk - 1) // chunk) * chunk
    rows_per_w = pad_bt // nw
    idx_d = jnp.pad(idx_d, (0, pad_bt - _BT))
    idx_w = jnp.pad(idx_w, (0, pad_bt - _BT))
    _C = 4 * _H

    def body(idx_d_hbm, idx_w_hbm, day_hbm, week_hbm, bcat_hbm,
             out_hbm, idxd_v, idxw_v, dayrows_v, weekrows_v,
             bcat_v, comb_v, sem_d, sem_w):
        nc = info.num_cores
        wid = lax.axis_index("s") * nc + lax.axis_index("c")
        base = wid * rows_per_w
        pltpu.sync_copy(idx_d_hbm.at[pl.ds(base, rows_per_w)], idxd_v)
        pltpu.sync_copy(idx_w_hbm.at[pl.ds(base, rows_per_w)], idxw_v)
        pltpu.sync_copy(bcat_hbm, bcat_v)
        pltpu.async_copy(day_hbm.at[idxd_v], dayrows_v, sem_d).wait()
        pltpu.async_copy(week_hbm.at[idxw_v], weekrows_v, sem_w).wait()
        for r in range(rows_per_w):
            for c in range(_C // _LANES):
                sl = pl.ds(c * _LANES, _LANES)
                comb_v[r, sl] = bcat_v[sl] + dayrows_v[r, sl] + weekrows_v[r, sl]
        pltpu.sync_copy(comb_v, out_hbm.at[pl.ds(base, rows_per_w)])

    fn = functools.partial(
        pl.kernel,
        mesh=plsc.VectorSubcoreMesh(core_axis_name="c", subcore_axis_name="s"),
        out_type=jax.ShapeDtypeStruct((pad_bt, _C), jnp.float32),
        scratch_types=[
            pltpu.VMEM((rows_per_w,), jnp.int32),
            pltpu.VMEM((rows_per_w,), jnp.int32),
            pltpu.VMEM((rows_per_w, _C), jnp.float32),
            pltpu.VMEM((rows_per_w, _C), jnp.float32),
            pltpu.VMEM((_C,), jnp.float32),
            pltpu.VMEM((rows_per_w, _C), jnp.float32),
            pltpu.SemaphoreType.DMA,
            pltpu.SemaphoreType.DMA,
        ],
    )(body)
    return fn(idx_d, idx_w, day_pad, week_pad, b_cat)


_G = 8


def _tc_body(x_ref, w_ref, b_ref, o_ref):
    for g in range(_G):
        xt = x_ref[g]
        h = lax.dot_general(xt, w_ref[...], (((0,), (0,)), ((), ())),
                            preferred_element_type=jnp.float32)
        v = h + b_ref[g]
        lane = lax.broadcasted_iota(jnp.int32, v.shape, 1)
        o_ref[g] = jnp.where(lane >= 3 * _H, jnp.maximum(v, 0.0), v)


def _tc_fused(xt, w_cat, bias3):
    return pl.pallas_call(
        _tc_body,
        grid=(_BT // _G,),
        in_specs=[
            pl.BlockSpec((_G, _DIN, _N), lambda i: (i, 0, 0)),
            pl.BlockSpec((_DIN, 4 * _H), lambda i: (0, 0)),
            pl.BlockSpec((_G, 1, 4 * _H), lambda i: (i, 0, 0)),
        ],
        out_specs=pl.BlockSpec((_G, _N, 4 * _H), lambda i: (i, 0, 0)),
        out_shape=jax.ShapeDtypeStruct((_BT, _N, 4 * _H), jnp.float32),
    )(xt, w_cat, bias3)


def kernel(x, t_day, t_week, W_tok, b_tok, day_table, week_table, W_sp, b_sp):
    idx_d = t_day.reshape(-1).astype(jnp.int32)
    idx_w = t_week.reshape(-1).astype(jnp.int32)
    day_pad = (jnp.zeros((day_table.shape[0], 4 * _H), jnp.float32)
               .at[:, _H : 2 * _H].set(day_table))
    week_pad = (jnp.zeros((week_table.shape[0], 4 * _H), jnp.float32)
                .at[:, 2 * _H : 3 * _H].set(week_table))
    b_cat = jnp.concatenate([b_tok, jnp.zeros((2 * _H,), jnp.float32), b_sp])
    bias = _sc_bias_rows(idx_d, idx_w, day_pad, week_pad, b_cat)
    bias3 = bias[:_BT].reshape(_BT, 1, 4 * _H)
    w_cat = (jnp.zeros((_DIN, 4 * _H), jnp.float32)
             .at[:, : _H].set(W_tok.T)
             .at[:, 3 * _H :].set(W_sp.T))
    xt = jnp.swapaxes(x.reshape(_BT, _N, _DIN), 1, 2)
    out = _tc_fused(xt, w_cat, bias3)
    return out.reshape(_B, _T, _N, 4 * _H)


# speedup vs baseline: 3.3146x; 1.0182x over previous
import functools

import jax
import jax.numpy as jnp
from jax import lax
from jax.experimental import pallas as pl
from jax.experimental.pallas import tpu as pltpu
from jax.experimental.pallas import tpu_sc as plsc

_B, _T, _N, _DIN, _H = 16, 12, 2048, 3, 32
_BT = _B * _T
_LANES = 16


def _sc_bias_rows(idx_d, idx_w, day_pad, week_pad, b_cat):
    info = plsc.get_sparse_core_info()
    nw = info.num_cores * info.num_subcores
    chunk = 8 * nw
    pad_bt = ((_BT + chunk - 1) // chunk) * chunk
    rows_per_w = pad_bt // nw
    idx_d = jnp.pad(idx_d, (0, pad_bt - _BT))
    idx_w = jnp.pad(idx_w, (0, pad_bt - _BT))
    _C = 4 * _H

    def body(idx_d_hbm, idx_w_hbm, day_hbm, week_hbm, bcat_hbm,
             out_hbm, idxd_v, idxw_v, dayrows_v, weekrows_v,
             bcat_v, comb_v, sem_d, sem_w):
        nc = info.num_cores
        wid = lax.axis_index("s") * nc + lax.axis_index("c")
        base = wid * rows_per_w
        pltpu.sync_copy(idx_d_hbm.at[pl.ds(base, rows_per_w)], idxd_v)
        pltpu.sync_copy(idx_w_hbm.at[pl.ds(base, rows_per_w)], idxw_v)
        pltpu.sync_copy(bcat_hbm, bcat_v)
        pltpu.async_copy(day_hbm.at[idxd_v], dayrows_v, sem_d).wait()
        pltpu.async_copy(week_hbm.at[idxw_v], weekrows_v, sem_w).wait()
        for r in range(rows_per_w):
            for c in range(_C // _LANES):
                sl = pl.ds(c * _LANES, _LANES)
                comb_v[r, sl] = bcat_v[sl] + dayrows_v[r, sl] + weekrows_v[r, sl]
        pltpu.sync_copy(comb_v, out_hbm.at[pl.ds(base, rows_per_w)])

    fn = functools.partial(
        pl.kernel,
        mesh=plsc.VectorSubcoreMesh(core_axis_name="c", subcore_axis_name="s"),
        out_type=jax.ShapeDtypeStruct((pad_bt, _C), jnp.float32),
        scratch_types=[
            pltpu.VMEM((rows_per_w,), jnp.int32),
            pltpu.VMEM((rows_per_w,), jnp.int32),
            pltpu.VMEM((rows_per_w, _C), jnp.float32),
            pltpu.VMEM((rows_per_w, _C), jnp.float32),
            pltpu.VMEM((_C,), jnp.float32),
            pltpu.VMEM((rows_per_w, _C), jnp.float32),
            pltpu.SemaphoreType.DMA,
            pltpu.SemaphoreType.DMA,
        ],
    )(body)
    return fn(idx_d, idx_w, day_pad, week_pad, b_cat)


_G = 16


def _tc_body(x_ref, w_ref, b_ref, o_ref):
    for g in range(_G):
        xt = x_ref[g]
        h = lax.dot_general(xt, w_ref[...], (((0,), (0,)), ((), ())),
                            preferred_element_type=jnp.float32)
        v = h + b_ref[g]
        lane = lax.broadcasted_iota(jnp.int32, v.shape, 1)
        o_ref[g] = jnp.where(lane >= 3 * _H, jnp.maximum(v, 0.0), v)


def _tc_fused(xt, w_cat, bias3):
    return pl.pallas_call(
        _tc_body,
        grid=(_BT // _G,),
        in_specs=[
            pl.BlockSpec((_G, _DIN, _N), lambda i: (i, 0, 0)),
            pl.BlockSpec((_DIN, 4 * _H), lambda i: (0, 0)),
            pl.BlockSpec((_G, 1, 4 * _H), lambda i: (i, 0, 0)),
        ],
        out_specs=pl.BlockSpec((_G, _N, 4 * _H), lambda i: (i, 0, 0)),
        out_shape=jax.ShapeDtypeStruct((_BT, _N, 4 * _H), jnp.float32),
    )(xt, w_cat, bias3)


def kernel(x, t_day, t_week, W_tok, b_tok, day_table, week_table, W_sp, b_sp):
    idx_d = t_day.reshape(-1).astype(jnp.int32)
    idx_w = t_week.reshape(-1).astype(jnp.int32)
    day_pad = (jnp.zeros((day_table.shape[0], 4 * _H), jnp.float32)
               .at[:, _H : 2 * _H].set(day_table))
    week_pad = (jnp.zeros((week_table.shape[0], 4 * _H), jnp.float32)
                .at[:, 2 * _H : 3 * _H].set(week_table))
    b_cat = jnp.concatenate([b_tok, jnp.zeros((2 * _H,), jnp.float32), b_sp])
    bias = _sc_bias_rows(idx_d, idx_w, day_pad, week_pad, b_cat)
    bias3 = bias[:_BT].reshape(_BT, 1, 4 * _H)
    w_cat = (jnp.zeros((_DIN, 4 * _H), jnp.float32)
             .at[:, : _H].set(W_tok.T)
             .at[:, 3 * _H :].set(W_sp.T))
    xt = jnp.swapaxes(x.reshape(_BT, _N, _DIN), 1, 2)
    out = _tc_fused(xt, w_cat, bias3)
    return out.reshape(_B, _T, _N, 4 * _H)
